# Initial kernel scaffold; baseline (speedup 1.0000x reference)
import jax
import jax.numpy as jnp
from jax.experimental import pallas as pl


def kernel(bucket_ids, table, W1, b1, W2, b2):
    raise NotImplementedError("write your pallas kernel here")


# speedup vs baseline: 1.8537x; 1.8537x over previous
import functools

import jax
import jax.numpy as jnp
from jax import lax
from jax.experimental import pallas as pl
from jax.experimental.pallas import tpu as pltpu
from jax.experimental.pallas import tpu_sc as plsc

B = 16384
N_HASHES = 26
EMB = 32
HID = 128

NC = 2
NS = 16
NW = NC * NS
LANES = 16

SPW = B // NW
CS = 64
NCHUNK = SPW // CS
ROWS_PER_CHUNK = CS * N_HASHES
DMA_ROWS = 128
NDMA = ROWS_PER_CHUNK // DMA_ROWS
IDX_ROWS_PER_WORKER = SPW * N_HASHES // DMA_ROWS


def _sc_gather_pool(idx2d, table):
  mesh = plsc.VectorSubcoreMesh(core_axis_name="c", subcore_axis_name="s")

  @functools.partial(
      pl.kernel,
      out_type=jax.ShapeDtypeStruct((B, EMB), jnp.float32),
      mesh=mesh,
      scratch_types=[
          pltpu.VMEM((IDX_ROWS_PER_WORKER, DMA_ROWS), jnp.int32),
          pltpu.VMEM((ROWS_PER_CHUNK, EMB), jnp.float32),
          pltpu.VMEM((CS, EMB), jnp.float32),
          pltpu.SemaphoreType.DMA,
      ],
      compiler_params=pltpu.CompilerParams(use_tc_tiling_on_sc=False),
  )
  def k(idx_hbm, table_hbm, out_hbm, idx_v, rows_v, pooled_v, sem):
    wid = lax.axis_index("s") * NC + lax.axis_index("c")
    pltpu.sync_copy(idx_hbm.at[pl.ds(wid * IDX_ROWS_PER_WORKER,
                                     IDX_ROWS_PER_WORKER)], idx_v)

    def chunk_body(ch, _):
      copies = []
      for j in range(NDMA):
        c = pltpu.async_copy(
            table_hbm.at[idx_v.at[ch * NDMA + j]],
            rows_v.at[pl.ds(j * DMA_ROWS, DMA_ROWS)],
            sem)
        copies.append(c)
      for c in copies:
        c.wait()

      def pool_body(i, _):
        r0 = i * N_HASHES
        for ph in range(2):
          sl = pl.ds(ph * LANES, LANES)
          acc = rows_v[r0, sl]
          for h in range(1, N_HASHES):
            acc = acc + rows_v[r0 + h, sl]
          pooled_v[i, sl] = acc
        return 0

      lax.fori_loop(0, CS, pool_body, 0)
      pltpu.sync_copy(pooled_v,
                      out_hbm.at[pl.ds(wid * SPW + ch * CS, CS)])
      return 0

    lax.fori_loop(0, NCHUNK, chunk_body, 0)

  return k(idx2d, table)


def _mlp_block(e_ref, w1_ref, b1_ref, w2_ref, b2_ref, o_ref):
  e = e_ref[...]
  h = jnp.dot(e, w1_ref[...], preferred_element_type=jnp.float32)
  h = jnp.maximum(h + b1_ref[...], 0.0)
  o_ref[...] = jnp.dot(h, w2_ref[...],
                       preferred_element_type=jnp.float32) + b2_ref[...]


def _tc_mlp(pooled, W1s, b1, W2, b2):
  BLK = 2048
  grid = (B // BLK,)
  return pl.pallas_call(
      _mlp_block,
      out_shape=jax.ShapeDtypeStruct((B, 1), jnp.float32),
      grid=grid,
      in_specs=[
          pl.BlockSpec((BLK, EMB), lambda i: (i, 0)),
          pl.BlockSpec((EMB, HID), lambda i: (0, 0)),
          pl.BlockSpec((1, HID), lambda i: (0, 0)),
          pl.BlockSpec((HID, 1), lambda i: (0, 0)),
          pl.BlockSpec((1, 1), lambda i: (0, 0)),
      ],
      out_specs=pl.BlockSpec((BLK, 1), lambda i: (i, 0)),
  )(pooled, W1s, b1, W2, b2)


@jax.jit
def kernel(bucket_ids, table, W1, b1, W2, b2):
  idx2d = bucket_ids.astype(jnp.int32).reshape(-1, DMA_ROWS)
  pooled = _sc_gather_pool(idx2d, table)
  W1s = (W1 * (1.0 / N_HASHES)).astype(jnp.float32)
  logit = _tc_mlp(pooled, W1s, b1.reshape(1, HID), W2, b2.reshape(1, 1))
  return logit[:, 0]
